# baseline (speedup 1.0000x reference)
import functools

import jax
import jax.numpy as jnp
from jax import lax
from jax.experimental import pallas as pl
from jax.experimental.pallas import tpu as pltpu
from jax.experimental.pallas import tpu_sc as plsc

_EMB = 64
_BATCH = 16384
_NC = 2
_NS = 16
_NW = _NC * _NS
_BPW = _BATCH // _NW
_GROUPS = _BPW // 16


def _body(tw, cw, tt, tc, out, idx_t, idx_c, col_t, col_c, accv, sem):
    wid = lax.axis_index("s") * _NC + lax.axis_index("c")
    base = wid * _BPW

    pltpu.sync_copy(tw.at[pl.ds(base, _BPW)], idx_t)
    pltpu.sync_copy(cw.at[pl.ds(base, _BPW)], idx_c)

    copies = []
    for c in range(_EMB):
        copies.append(pltpu.async_copy(tt.at[c].at[idx_t], col_t.at[c], sem))
        copies.append(pltpu.async_copy(tc.at[c].at[idx_c], col_c.at[c], sem))
    for cp in copies:
        cp.wait()

    def grp_body(g, tot):
        sl = pl.ds(g * 16, 16)

        def feat_body(c, acc):
            return acc + col_t[c, sl] * col_c[c, sl]

        x = lax.fori_loop(0, _EMB, feat_body, jnp.zeros((16,), jnp.float32))

        u = jnp.exp(-jnp.abs(x))
        s = u / (u + 2.0)
        s2 = s * s
        poly = 1.0 + s2 * (1.0 / 3.0 + s2 * (1.0 / 5.0 + s2 * (
            1.0 / 7.0 + s2 * (1.0 / 9.0 + s2 * (1.0 / 11.0)))))
        return tot + jnp.maximum(-x, 0.0) + 2.0 * (s * poly)

    tot = lax.fori_loop(0, _GROUPS, grp_body, jnp.zeros((16,), jnp.float32))
    accv[...] = tot
    pltpu.sync_copy(accv, out.at[pl.ds(wid * 16, 16)])


@jax.jit
def _partials(tw, cw, tt, tc):
    mesh = plsc.VectorSubcoreMesh(core_axis_name="c", subcore_axis_name="s")
    run = pl.kernel(
        _body,
        mesh=mesh,
        compiler_params=pltpu.CompilerParams(
            needs_layout_passes=False, use_tc_tiling_on_sc=False),
        out_type=jax.ShapeDtypeStruct((_NW * 16,), jnp.float32),
        scratch_types=[
            pltpu.VMEM((_BPW,), jnp.int32),
            pltpu.VMEM((_BPW,), jnp.int32),
            pltpu.VMEM((_EMB, _BPW), jnp.float32),
            pltpu.VMEM((_EMB, _BPW), jnp.float32),
            pltpu.VMEM((16,), jnp.float32),
            pltpu.SemaphoreType.DMA,
        ],
    )
    return run(tw, cw, tt, tc)


def kernel(target_word, context_word, target_embeddings, context_embeddings):
    tw = target_word.astype(jnp.int32)
    cw = context_word.astype(jnp.int32)
    part = _partials(tw, cw, target_embeddings.T, context_embeddings.T)
    return jnp.sum(part)


# speedup vs baseline: 9.0549x; 9.0549x over previous
import functools

import jax
import jax.numpy as jnp
from jax import lax
from jax.experimental import pallas as pl
from jax.experimental.pallas import tpu as pltpu
from jax.experimental.pallas import tpu_sc as plsc

_V = 1_000_000
_EMB = 64
_B = 16384
_NC = 2
_NS = 16
_NW = _NC * _NS
_BPW = _B // _NW
_HALF = _BPW // 2
_CHUNK = 128


def _body(tw, cw, te2, ce2, out, idx_t, idx_c, row_t, row_c, cb_t, cb_c,
          rows_t, rows_c, accv, sem):
    wid = lax.axis_index("s") * _NC + lax.axis_index("c")
    base = wid * _BPW

    pltpu.sync_copy(tw.at[pl.ds(base, _BPW)], idx_t)
    pltpu.sync_copy(cw.at[pl.ds(base, _BPW)], idx_c)

    def pc(i, _):
        sl = pl.ds(i * 16, 16)
        row_t[sl] = lax.shift_right_logical(idx_t[sl], 1)
        row_c[sl] = lax.shift_right_logical(idx_c[sl], 1)
        cb_t[sl] = (idx_t[sl] & 1) * _EMB
        cb_c[sl] = (idx_c[sl] & 1) * _EMB
        return 0

    lax.fori_loop(0, _BPW // 16, pc, 0)

    lane = lax.iota(jnp.int32, 16)
    total = jnp.zeros((16,), jnp.float32)

    for h in range(2):
        hbase = h * _HALF
        copies = []
        for j in range(_HALF // _CHUNK):
            isl = pl.ds(hbase + j * _CHUNK, _CHUNK)
            dsl = pl.ds(j * _CHUNK, _CHUNK)
            copies.append(
                pltpu.async_copy(te2.at[row_t.at[isl]], rows_t.at[dsl], sem))
            copies.append(
                pltpu.async_copy(ce2.at[row_c.at[isl]], rows_c.at[dsl], sem))
        for cp in copies:
            cp.wait()

        def grp_body(g, tot):
            x = jnp.zeros((16,), jnp.float32)
            for k in range(16):
                r = g * 16 + k
                ct = cb_t[pl.ds(hbase + r, 16)][0]
                cc = cb_c[pl.ds(hbase + r, 16)][0]
                acc = (rows_t[r, pl.ds(ct, 16)]
                       * rows_c[r, pl.ds(cc, 16)])
                for cb in range(1, _EMB // 16):
                    acc = acc + (rows_t[r, pl.ds(ct + cb * 16, 16)]
                                 * rows_c[r, pl.ds(cc + cb * 16, 16)])
                x = jnp.where(lane == k, jnp.sum(acc), x)

            u = jnp.exp(-jnp.abs(x))
            s = u / (u + 2.0)
            s2 = s * s
            poly = 1.0 + s2 * (1.0 / 3.0 + s2 * (1.0 / 5.0 + s2 * (
                1.0 / 7.0 + s2 * (1.0 / 9.0 + s2 * (1.0 / 11.0)))))
            return tot + jnp.maximum(-x, 0.0) + 2.0 * (s * poly)

        total = lax.fori_loop(0, _HALF // 16, grp_body, total)

    accv[...] = total
    pltpu.sync_copy(accv, out.at[pl.ds(wid * 16, 16)])


@jax.jit
def _partials(tw, cw, te2, ce2):
    mesh = plsc.VectorSubcoreMesh(core_axis_name="c", subcore_axis_name="s")
    run = pl.kernel(
        _body,
        mesh=mesh,
        compiler_params=pltpu.CompilerParams(
            needs_layout_passes=False, use_tc_tiling_on_sc=True),
        out_type=jax.ShapeDtypeStruct((_NW * 16,), jnp.float32),
        scratch_types=[
            pltpu.VMEM((_BPW,), jnp.int32),
            pltpu.VMEM((_BPW,), jnp.int32),
            pltpu.VMEM((_BPW,), jnp.int32),
            pltpu.VMEM((_BPW,), jnp.int32),
            pltpu.VMEM((_BPW + 16,), jnp.int32),
            pltpu.VMEM((_BPW + 16,), jnp.int32),
            pltpu.VMEM((_HALF, 128), jnp.float32),
            pltpu.VMEM((_HALF, 128), jnp.float32),
            pltpu.VMEM((16,), jnp.float32),
            pltpu.SemaphoreType.DMA,
        ],
    )
    return run(tw, cw, te2, ce2)


def kernel(target_word, context_word, target_embeddings, context_embeddings):
    tw = target_word.astype(jnp.int32)
    cw = context_word.astype(jnp.int32)
    te2 = target_embeddings.reshape(_V // 2, 128)
    ce2 = context_embeddings.reshape(_V // 2, 128)
    part = _partials(tw, cw, te2, ce2)
    return jnp.sum(part)


# speedup vs baseline: 16.2350x; 1.7930x over previous
import functools

import jax
import jax.numpy as jnp
from jax import lax
from jax.experimental import pallas as pl
from jax.experimental.pallas import tpu as pltpu
from jax.experimental.pallas import tpu_sc as plsc

_V = 1_000_000
_EMB = 64
_B = 16384
_NC = 2
_NS = 16
_NW = _NC * _NS
_NBLK = 7813
_BASEB = _NBLK // _NW
_EXTRA = _NBLK % _NW
_CAP = _B


def _select(src_hbm, ibuf, selidx, selpos, bstart, bend, lane):
    def chunk_body(ci, base):
        pltpu.sync_copy(src_hbm.at[pl.ds(ci * 512, 512)], ibuf)

        def vec_body(i, b):
            sl = pl.ds(i * 16, 16)
            v = ibuf[sl]
            blk = lax.shift_right_logical(v, 7)
            m = (blk >= bstart) & (blk < bend)
            cnt = plsc.all_reduce_population_count(m)[0]

            @pl.when(cnt > 0)
            def _():
                poss = ci * 512 + i * 16 + lane
                plsc.store_compressed(selidx.at[pl.ds(b, 16)], v, mask=m)
                plsc.store_compressed(selpos.at[pl.ds(b, 16)], poss, mask=m)

            return b + cnt

        return lax.fori_loop(0, 32, vec_body, base)

    return lax.fori_loop(0, _B // 512, chunk_body, 0)


def _extract_block(b, vb, par, selidx, selpos, ntot, mq, mp, stage, out_hbm,
                   sem2, lane, carry, tail=False):
    issued0, drained0 = carry
    nvec = lax.shift_right_logical(ntot + 15, 4)

    def scan_body(i, c):
        issued, drained = c
        sl = pl.ds(i * 16, 16)
        v = selidx[sl]
        m = lax.shift_right_logical(v, 7) == b
        cnt = plsc.all_reduce_population_count(m)[0]

        def drain_all(d):
            def dr(j, _):
                pltpu.make_async_copy(out_hbm.at[pl.ds(0, _EMB)],
                                      stage.at[0], sem2).wait()
                return 0
            lax.fori_loop(0, issued - d, dr, 0)
            return issued

        drained = lax.cond((cnt > 0) & (issued - drained > 48),
                           drain_all, lambda d: d, drained)

        @pl.when(cnt > 0)
        def _():
            plsc.store_compressed(mq.at[pl.ds(0, 16)], v & 127, mask=m)
            plsc.store_compressed(mp.at[pl.ds(0, 16)], selpos[sl], mask=m)

            def ent_body(j, _):
                q = mq[pl.ds(j, 16)][0]
                pos = mp[pl.ds(j, 16)][0]
                slot = (issued + j) & 63
                for fg in range(4):
                    f16 = fg * 16 + lane
                    if tail:
                        vals = plsc.load_gather(
                            vb, [f16, jnp.full((16,), q, jnp.int32)])
                    else:
                        vals = plsc.load_gather(
                            vb, [jnp.full((16,), par, jnp.int32), f16,
                                 jnp.full((16,), q, jnp.int32)])
                    stage[slot, pl.ds(fg * 16, 16)] = vals
                pltpu.async_copy(stage.at[slot],
                                 out_hbm.at[pl.ds(pos * _EMB, _EMB)], sem2)
                return 0

            lax.fori_loop(0, cnt, ent_body, 0)

        return (issued + cnt, drained)

    return lax.fori_loop(0, nvec, scan_body, (issued0, drained0))


def _gather_body(tw, cw, tt, tcc, out_t, out_c, ibuf, selidx_t, selpos_t,
                 selidx_c, selpos_c, mq, mp, vb_t, vb_c, vtail_t, vtail_c,
                 stage, semp, sem2):
    core = lax.axis_index("c")
    sid = lax.axis_index("s")
    wid = sid * _NC + core
    lane = lax.iota(jnp.int32, 16)

    bstart = wid * _BASEB + jnp.minimum(wid, _EXTRA)
    nb = _BASEB + jnp.where(wid < _EXTRA, 1, 0)
    bend = bstart + nb
    nb_full = nb - jnp.where(wid == _NW - 1, 1, 0)

    ntot_t = _select(tw, ibuf, selidx_t, selpos_t, bstart, bend, lane)
    ntot_c = _select(cw, ibuf, selidx_c, selpos_c, bstart, bend, lane)

    def issue(b, par):
        sl = pl.ds(b * 128, 128)
        pltpu.async_copy(tt.at[:, sl], vb_t.at[par], semp)
        pltpu.async_copy(tcc.at[:, sl], vb_c.at[par], semp)

    def wait_pair():
        dummy2d = tt.at[:, pl.ds(0, 128)]
        pltpu.make_async_copy(dummy2d, vb_t.at[0], semp).wait()
        pltpu.make_async_copy(dummy2d, vb_c.at[0], semp).wait()

    issue(bstart, 0)

    def blk_body(r, carry):
        b = bstart + r
        par = lax.rem(r, 2)
        wait_pair()

        @pl.when(r + 1 < nb_full)
        def _():
            issue(b + 1, 1 - par)

        carry = _extract_block(b, vb_t, par, selidx_t, selpos_t, ntot_t, mq,
                               mp, stage, out_t, sem2, lane, carry)
        carry = _extract_block(b, vb_c, par, selidx_c, selpos_c, ntot_c, mq,
                               mp, stage, out_c, sem2, lane, carry)
        return carry

    carry = lax.fori_loop(0, nb_full, blk_body, (0, 0))

    issued, drained = carry

    @pl.when(wid == _NW - 1)
    def _():
        tsl = pl.ds((_NBLK - 1) * 128, _V - (_NBLK - 1) * 128)
        pltpu.sync_copy(tt.at[:, tsl], vtail_t)
        pltpu.sync_copy(tcc.at[:, tsl], vtail_c)
        c2 = _extract_block(_NBLK - 1, vtail_t, 0, selidx_t, selpos_t,
                            ntot_t, mq, mp, stage, out_t, sem2, lane,
                            (issued, drained), tail=True)
        c3 = _extract_block(_NBLK - 1, vtail_c, 0, selidx_c, selpos_c,
                            ntot_c, mq, mp, stage, out_c, sem2, lane, c2,
                            tail=True)

        def dr(j, _):
            pltpu.make_async_copy(out_t.at[pl.ds(0, _EMB)], stage.at[0],
                                  sem2).wait()
            return 0

        lax.fori_loop(0, c3[0] - c3[1], dr, 0)

    @pl.when(wid != _NW - 1)
    def _():
        def dr(j, _):
            pltpu.make_async_copy(out_t.at[pl.ds(0, _EMB)], stage.at[0],
                                  sem2).wait()
            return 0

        lax.fori_loop(0, issued - drained, dr, 0)


def _compute_body(gt, gc, out, vt, vc, accv):
    wid = lax.axis_index("s") * _NC + lax.axis_index("c")
    pltpu.sync_copy(gt.at[pl.ds(wid * 32768, 32768)], vt)
    pltpu.sync_copy(gc.at[pl.ds(wid * 32768, 32768)], vc)
    lane = lax.iota(jnp.int32, 16)

    def grp_body(g, tot):
        x = jnp.zeros((16,), jnp.float32)
        for k in range(16):
            off = (g * 16 + k) * _EMB
            acc = vt[pl.ds(off, 16)] * vc[pl.ds(off, 16)]
            for cb in range(1, _EMB // 16):
                acc = acc + (vt[pl.ds(off + cb * 16, 16)]
                             * vc[pl.ds(off + cb * 16, 16)])
            x = jnp.where(lane == k, jnp.sum(acc), x)

        u = jnp.exp(-jnp.abs(x))
        s = u / (u + 2.0)
        s2 = s * s
        poly = 1.0 + s2 * (1.0 / 3.0 + s2 * (1.0 / 5.0 + s2 * (
            1.0 / 7.0 + s2 * (1.0 / 9.0 + s2 * (1.0 / 11.0)))))
        return tot + jnp.maximum(-x, 0.0) + 2.0 * (s * poly)

    tot = lax.fori_loop(0, 32, grp_body, jnp.zeros((16,), jnp.float32))
    accv[...] = tot
    pltpu.sync_copy(accv, out.at[pl.ds(wid * 16, 16)])


@jax.jit
def _run(tw, cw, tt, tcc):
    mesh = plsc.VectorSubcoreMesh(core_axis_name="c", subcore_axis_name="s")
    params = pltpu.CompilerParams(
        needs_layout_passes=False, use_tc_tiling_on_sc=True)

    gather = pl.kernel(
        _gather_body,
        mesh=mesh,
        compiler_params=params,
        out_type=(
            jax.ShapeDtypeStruct((_B * _EMB,), jnp.float32),
            jax.ShapeDtypeStruct((_B * _EMB,), jnp.float32),
        ),
        scratch_types=[
            pltpu.VMEM((512,), jnp.int32),
            pltpu.VMEM((_CAP + 16,), jnp.int32),
            pltpu.VMEM((_CAP + 16,), jnp.int32),
            pltpu.VMEM((_CAP + 16,), jnp.int32),
            pltpu.VMEM((_CAP + 16,), jnp.int32),
            pltpu.VMEM((32,), jnp.int32),
            pltpu.VMEM((32,), jnp.int32),
            pltpu.VMEM((2, _EMB, 128), jnp.float32),
            pltpu.VMEM((2, _EMB, 128), jnp.float32),
            pltpu.VMEM((_EMB, 64), jnp.float32),
            pltpu.VMEM((_EMB, 64), jnp.float32),
            pltpu.VMEM((64, _EMB), jnp.float32),
            pltpu.SemaphoreType.DMA,
            pltpu.SemaphoreType.DMA,
        ],
    )
    gt, gc = gather(tw, cw, tt, tcc)

    compute = pl.kernel(
        _compute_body,
        mesh=mesh,
        compiler_params=params,
        out_type=jax.ShapeDtypeStruct((_NW * 16,), jnp.float32),
        scratch_types=[
            pltpu.VMEM((32768,), jnp.float32),
            pltpu.VMEM((32768,), jnp.float32),
            pltpu.VMEM((16,), jnp.float32),
        ],
    )
    return compute(gt, gc)


def kernel(target_word, context_word, target_embeddings, context_embeddings):
    tw = target_word.astype(jnp.int32)
    cw = context_word.astype(jnp.int32)
    part = _run(tw, cw, target_embeddings.T, context_embeddings.T)
    return jnp.sum(part)
